# baseline (speedup 1.0000x reference)
import functools

import jax
import jax.numpy as jnp
from jax import lax
from jax.experimental import pallas as pl
from jax.experimental.pallas import tpu as pltpu
from jax.experimental.pallas import tpu_sc as plsc

BT = 128
BI = 512


@functools.lru_cache(maxsize=None)
def _make_row_gather(V, D, B):
    info = plsc.get_sparse_core_info()
    NC, NS = info.num_cores, info.num_subcores
    NW = NC * NS
    assert B % NW == 0
    b_per_w = B // NW
    C = next(c for c in (96, 80, 64, 48, 32, 16, 8) if b_per_w % c == 0)
    n_chunks = b_per_w // C
    mesh = plsc.VectorSubcoreMesh(core_axis_name="c", subcore_axis_name="s")

    @functools.partial(
        pl.kernel,
        mesh=mesh,
        out_type=jax.ShapeDtypeStruct((B, D), jnp.float32),
        scratch_types=[
            pltpu.VMEM((C,), jnp.int32),
            pltpu.VMEM((C, D), jnp.float32),
            pltpu.SemaphoreType.DMA,
        ],
    )
    def gather(table_hbm, idx_hbm, out_hbm, idx_v, rows_v, sem):
        wid = lax.axis_index("s") * NC + lax.axis_index("c")
        base = wid * b_per_w
        for c in range(n_chunks):
            off = base + c * C
            pltpu.sync_copy(idx_hbm.at[pl.ds(off, C)], idx_v)
            pltpu.async_copy(table_hbm.at[idx_v], rows_v, sem).wait()
            pltpu.sync_copy(rows_v, out_hbm.at[pl.ds(off, C)])

    return gather


def _make_ffn_body(E, NJ):
    def body(ts_ref, x_hbm, w1_ref, w3_ref, w2_ref, y_hbm,
             xv_ref, yv_ref, sem):
        e = pl.program_id(0)
        j = pl.program_id(1)
        dn = (((1,), (1,)), ((), ()))

        @pl.when((e == 0) & (j == 0))
        def _():
            pltpu.make_async_copy(x_hbm, xv_ref, sem).start()
            pltpu.make_async_copy(x_hbm, xv_ref, sem).wait()

        t0 = ts_ref[e]
        t1 = ts_ref[e + 1]

        def tile_step(t, _):
            rows = xv_ref[pl.ds(t * BT, BT)]
            h1 = lax.dot_general(rows, w1_ref[0], dn,
                                 preferred_element_type=jnp.float32)
            h3 = lax.dot_general(rows, w3_ref[0], dn,
                                 preferred_element_type=jnp.float32)
            act = h1 * jax.nn.sigmoid(h1) * h3
            part = lax.dot_general(act, w2_ref[0], dn,
                                   preferred_element_type=jnp.float32)

            @pl.when(j == 0)
            def _():
                yv_ref[pl.ds(t * BT, BT)] = part

            @pl.when(j > 0)
            def _():
                yv_ref[pl.ds(t * BT, BT)] = yv_ref[pl.ds(t * BT, BT)] + part

            return 0

        lax.fori_loop(t0, t1, tile_step, 0)

        @pl.when((e == E - 1) & (j == NJ - 1))
        def _():
            pltpu.make_async_copy(yv_ref, y_hbm, sem).start()
            pltpu.make_async_copy(yv_ref, y_hbm, sem).wait()

    return body


def kernel(x, expert_indices, w13, w2):
    T, D = x.shape
    A = expert_indices.shape[1]
    E = w13.shape[0]
    I = w2.shape[2]
    N = T * A
    NJ = I // BI
    MAX_TILES = N // BT + E
    NP = MAX_TILES * BT

    idx_flat = expert_indices.reshape(N).astype(jnp.int32)
    counts = jnp.bincount(idx_flat, length=E).astype(jnp.int32)
    tiles_per_e = (counts + BT - 1) // BT
    cum_tiles = jnp.cumsum(tiles_per_e)
    padded_start = (cum_tiles - tiles_per_e) * BT
    orig_start = jnp.cumsum(counts) - counts
    order = jnp.argsort(idx_flat, stable=True)
    sorted_e = idx_flat[order]
    dest_row = padded_start[sorted_e] + (jnp.arange(N, dtype=jnp.int32)
                                         - orig_start[sorted_e])
    src_tok = (jnp.arange(NP, dtype=jnp.int32) % T).at[dest_row].set(
        (order // A).astype(jnp.int32))
    inv_row = jnp.zeros((N,), jnp.int32).at[order].set(dest_row)
    tile_starts = jnp.concatenate(
        [jnp.zeros((1,), jnp.int32), cum_tiles.astype(jnp.int32)])

    x_s = _make_row_gather(T, D, NP)(x, src_tok)

    grid_spec = pltpu.PrefetchScalarGridSpec(
        num_scalar_prefetch=1,
        grid=(E, NJ),
        in_specs=[
            pl.BlockSpec(memory_space=pl.ANY),
            pl.BlockSpec((1, BI, D), lambda e, j, ts: (e, j, 0)),
            pl.BlockSpec((1, BI, D), lambda e, j, ts: (e, NJ + j, 0)),
            pl.BlockSpec((1, D, BI), lambda e, j, ts: (e, 0, j)),
        ],
        out_specs=pl.BlockSpec(memory_space=pl.ANY),
        scratch_shapes=[
            pltpu.VMEM((NP, D), jnp.float32),
            pltpu.VMEM((NP, D), jnp.float32),
            pltpu.SemaphoreType.DMA,
        ],
    )
    y_s = pl.pallas_call(
        _make_ffn_body(E, NJ),
        grid_spec=grid_spec,
        out_shape=jax.ShapeDtypeStruct((NP, D), jnp.float32),
        compiler_params=pltpu.CompilerParams(
            dimension_semantics=("arbitrary", "arbitrary")),
    )(tile_starts, x_s, w13, w13, w2)

    out_flat = _make_row_gather(NP, D, N)(y_s, inv_row)
    return out_flat.reshape(T, A, D)


# speedup vs baseline: 1.2584x; 1.2584x over previous
import functools

import jax
import jax.numpy as jnp
from jax import lax
from jax.experimental import pallas as pl
from jax.experimental.pallas import tpu as pltpu
from jax.experimental.pallas import tpu_sc as plsc

BT = 512


@functools.lru_cache(maxsize=None)
def _make_row_gather(V, D, B):
    info = plsc.get_sparse_core_info()
    NC, NS = info.num_cores, info.num_subcores
    NW = NC * NS
    assert B % NW == 0
    b_per_w = B // NW
    C = next(c for c in (96, 80, 64, 48, 32, 16, 8) if b_per_w % c == 0)
    n_chunks = b_per_w // C
    mesh = plsc.VectorSubcoreMesh(core_axis_name="c", subcore_axis_name="s")

    @functools.partial(
        pl.kernel,
        mesh=mesh,
        out_type=jax.ShapeDtypeStruct((B, D), jnp.float32),
        scratch_types=[
            pltpu.VMEM((C,), jnp.int32),
            pltpu.VMEM((C, D), jnp.float32),
            pltpu.SemaphoreType.DMA,
        ],
    )
    def gather(table_hbm, idx_hbm, out_hbm, idx_v, rows_v, sem):
        wid = lax.axis_index("s") * NC + lax.axis_index("c")
        base = wid * b_per_w
        for c in range(n_chunks):
            off = base + c * C
            pltpu.sync_copy(idx_hbm.at[pl.ds(off, C)], idx_v)
            pltpu.async_copy(table_hbm.at[idx_v], rows_v, sem).wait()
            pltpu.sync_copy(rows_v, out_hbm.at[pl.ds(off, C)])

    return gather


def _act_body(eid_ref, nt_ref, x_ref, w13_ref, act_ref, *, I):
    t = pl.program_id(0)

    @pl.when(t < nt_ref[0])
    def _():
        xv = x_ref[...]
        dn = (((1,), (1,)), ((), ()))
        h1 = lax.dot_general(xv, w13_ref[0, :I, :], dn,
                             preferred_element_type=jnp.float32)
        h3 = lax.dot_general(xv, w13_ref[0, I:, :], dn,
                             preferred_element_type=jnp.float32)
        act_ref[...] = (h1 * jax.nn.sigmoid(h1) * h3).astype(jnp.bfloat16)


def _y_body(eid_ref, nt_ref, act_ref, w2_ref, y_ref):
    t = pl.program_id(0)

    @pl.when(t < nt_ref[0])
    def _():
        act = act_ref[...].astype(jnp.float32)
        dn = (((1,), (1,)), ((), ()))
        y_ref[...] = lax.dot_general(act, w2_ref[0], dn,
                                     preferred_element_type=jnp.float32)


def kernel(x, expert_indices, w13, w2):
    T, D = x.shape
    A = expert_indices.shape[1]
    E = w13.shape[0]
    I = w2.shape[2]
    N = T * A
    MAX_TILES = N // BT + E
    NP = MAX_TILES * BT

    idx_flat = expert_indices.reshape(N).astype(jnp.int32)
    counts = jnp.bincount(idx_flat, length=E).astype(jnp.int32)
    tiles_per_e = (counts + BT - 1) // BT
    cum_tiles = jnp.cumsum(tiles_per_e)
    padded_start = (cum_tiles - tiles_per_e) * BT
    orig_start = jnp.cumsum(counts) - counts
    order = jnp.argsort(idx_flat, stable=True)
    sorted_e = idx_flat[order]
    dest_row = padded_start[sorted_e] + (jnp.arange(N, dtype=jnp.int32)
                                         - orig_start[sorted_e])
    src_tok = (jnp.arange(NP, dtype=jnp.int32) % T).at[dest_row].set(
        (order // A).astype(jnp.int32))
    inv_row = jnp.zeros((N,), jnp.int32).at[order].set(dest_row)
    tile_eid = jnp.minimum(
        jnp.searchsorted(cum_tiles, jnp.arange(MAX_TILES, dtype=jnp.int32),
                         side="right"),
        E - 1).astype(jnp.int32)
    num_tiles = cum_tiles[-1].astype(jnp.int32).reshape(1)

    x_s = _make_row_gather(T, D, NP)(x, src_tok)

    def _tmap(t, eid, nt):
        return (jnp.minimum(t, nt[0] - 1), 0)

    def _wmap(t, eid, nt):
        return (eid[jnp.minimum(t, nt[0] - 1)], 0, 0)

    act = pl.pallas_call(
        functools.partial(_act_body, I=I),
        grid_spec=pltpu.PrefetchScalarGridSpec(
            num_scalar_prefetch=2,
            grid=(MAX_TILES,),
            in_specs=[
                pl.BlockSpec((BT, D), _tmap),
                pl.BlockSpec((1, 2 * I, D), _wmap),
            ],
            out_specs=pl.BlockSpec((BT, I), _tmap),
        ),
        out_shape=jax.ShapeDtypeStruct((NP, I), jnp.bfloat16),
        compiler_params=pltpu.CompilerParams(
            dimension_semantics=("arbitrary",)),
    )(tile_eid, num_tiles, x_s, w13)

    y_s = pl.pallas_call(
        _y_body,
        grid_spec=pltpu.PrefetchScalarGridSpec(
            num_scalar_prefetch=2,
            grid=(MAX_TILES,),
            in_specs=[
                pl.BlockSpec((BT, I), _tmap),
                pl.BlockSpec((1, D, I), _wmap),
            ],
            out_specs=pl.BlockSpec((BT, D), _tmap),
        ),
        out_shape=jax.ShapeDtypeStruct((NP, D), jnp.float32),
        compiler_params=pltpu.CompilerParams(
            dimension_semantics=("arbitrary",)),
    )(tile_eid, num_tiles, act, w2)

    out_flat = _make_row_gather(NP, D, N)(y_s, inv_row)
    return out_flat.reshape(T, A, D)


# speedup vs baseline: 1.3589x; 1.0798x over previous
import functools

import jax
import jax.numpy as jnp
from jax import lax
from jax.experimental import pallas as pl
from jax.experimental.pallas import tpu as pltpu
from jax.experimental.pallas import tpu_sc as plsc

BT = 512


@functools.lru_cache(maxsize=None)
def _make_row_gather(V, D, B):
    info = plsc.get_sparse_core_info()
    NC, NS = info.num_cores, info.num_subcores
    NW = NC * NS
    assert B % NW == 0
    b_per_w = B // NW
    C = next(c for c in (96, 80, 64, 48, 32, 16, 8) if b_per_w % c == 0)
    n_chunks = b_per_w // C
    mesh = plsc.VectorSubcoreMesh(core_axis_name="c", subcore_axis_name="s")

    @functools.partial(
        pl.kernel,
        mesh=mesh,
        out_type=jax.ShapeDtypeStruct((B, D), jnp.float32),
        scratch_types=[
            pltpu.VMEM((C,), jnp.int32),
            pltpu.VMEM((C, D), jnp.float32),
            pltpu.SemaphoreType.DMA,
        ],
    )
    def gather(table_hbm, idx_hbm, out_hbm, idx_v, rows_v, sem):
        wid = lax.axis_index("s") * NC + lax.axis_index("c")
        base = wid * b_per_w
        for c in range(n_chunks):
            off = base + c * C
            pltpu.sync_copy(idx_hbm.at[pl.ds(off, C)], idx_v)
            pltpu.async_copy(table_hbm.at[idx_v], rows_v, sem).wait()
            pltpu.sync_copy(rows_v, out_hbm.at[pl.ds(off, C)])

    return gather


def _ffn_body(eid_ref, nt_ref, x_ref, w13_ref, w2_ref, out_ref, *, I):
    t = pl.program_id(0)

    @pl.when(t < nt_ref[0])
    def _():
        xv = x_ref[...]
        dn = (((1,), (1,)), ((), ()))
        h1 = lax.dot_general(xv, w13_ref[0, :I, :], dn,
                             preferred_element_type=jnp.float32)
        h3 = lax.dot_general(xv, w13_ref[0, I:, :], dn,
                             preferred_element_type=jnp.float32)
        act = h1 * jax.nn.sigmoid(h1) * h3
        out_ref[...] = lax.dot_general(act, w2_ref[0], dn,
                                       preferred_element_type=jnp.float32)


def kernel(x, expert_indices, w13, w2):
    T, D = x.shape
    A = expert_indices.shape[1]
    E = w13.shape[0]
    I = w2.shape[2]
    N = T * A
    MAX_TILES = N // BT + E
    NP = MAX_TILES * BT

    idx_flat = expert_indices.reshape(N).astype(jnp.int32)
    counts = jnp.bincount(idx_flat, length=E).astype(jnp.int32)
    tiles_per_e = (counts + BT - 1) // BT
    cum_tiles = jnp.cumsum(tiles_per_e)
    padded_start = (cum_tiles - tiles_per_e) * BT
    orig_start = jnp.cumsum(counts) - counts
    order = jnp.argsort(idx_flat, stable=True)
    sorted_e = idx_flat[order]
    dest_row = padded_start[sorted_e] + (jnp.arange(N, dtype=jnp.int32)
                                         - orig_start[sorted_e])
    src_tok = (jnp.arange(NP, dtype=jnp.int32) % T).at[dest_row].set(
        (order // A).astype(jnp.int32))
    inv_row = jnp.zeros((N,), jnp.int32).at[order].set(dest_row)
    tile_eid = jnp.minimum(
        jnp.searchsorted(cum_tiles, jnp.arange(MAX_TILES, dtype=jnp.int32),
                         side="right"),
        E - 1).astype(jnp.int32)
    num_tiles = cum_tiles[-1].astype(jnp.int32).reshape(1)

    x_s = _make_row_gather(T, D, NP)(x, src_tok)

    def _tmap(t, eid, nt):
        return (jnp.minimum(t, nt[0] - 1), 0)

    def _wmap(t, eid, nt):
        return (eid[jnp.minimum(t, nt[0] - 1)], 0, 0)

    y_s = pl.pallas_call(
        functools.partial(_ffn_body, I=I),
        grid_spec=pltpu.PrefetchScalarGridSpec(
            num_scalar_prefetch=2,
            grid=(MAX_TILES,),
            in_specs=[
                pl.BlockSpec((BT, D), _tmap),
                pl.BlockSpec((1, 2 * I, D), _wmap),
                pl.BlockSpec((1, D, I), _wmap),
            ],
            out_specs=pl.BlockSpec((BT, D), _tmap),
        ),
        out_shape=jax.ShapeDtypeStruct((NP, D), jnp.float32),
        compiler_params=pltpu.CompilerParams(
            dimension_semantics=("arbitrary",),
            vmem_limit_bytes=100 * 1024 * 1024),
    )(tile_eid, num_tiles, x_s, w13, w2)

    out_flat = _make_row_gather(NP, D, N)(y_s, inv_row)
    return out_flat.reshape(T, A, D)
